# baseline (speedup 1.0000x reference)
import functools

import jax
import jax.numpy as jnp
from jax import lax
from jax.experimental import pallas as pl
from jax.experimental.pallas import tpu as pltpu
from jax.experimental.pallas import tpu_sc as plsc

BATCH = 16384
WORD_DIM = 128
URL_DIM = 16
AUTHOR_DIM = 16

_NC = 2
_NS = 16
_NW = _NC * _NS
_ROWS_PER_TILE = BATCH // _NW
_CHUNK = 128
_NCHUNK = _ROWS_PER_TILE // _CHUNK


def _sc_gather_body(url_table, author_table, url_idx, author_idx,
                    url_out, author_out,
                    uidx_v, aidx_v, ubuf, abuf, sem):
    wid = lax.axis_index("s") * _NC + lax.axis_index("c")
    base = wid * _ROWS_PER_TILE
    pltpu.sync_copy(url_idx.at[pl.ds(base, _ROWS_PER_TILE)], uidx_v)
    pltpu.sync_copy(author_idx.at[pl.ds(base, _ROWS_PER_TILE)], aidx_v)
    handles = []
    for j in range(_NCHUNK):
        sl = pl.ds(j * _CHUNK, _CHUNK)
        handles.append(pltpu.async_copy(url_table.at[uidx_v.at[sl]], ubuf.at[sl], sem))
        handles.append(pltpu.async_copy(author_table.at[aidx_v.at[sl]], abuf.at[sl], sem))
    for h in handles:
        h.wait()
    pltpu.sync_copy(ubuf, url_out.at[pl.ds(base, _ROWS_PER_TILE)])
    pltpu.sync_copy(abuf, author_out.at[pl.ds(base, _ROWS_PER_TILE)])


@functools.cache
def _sc_gather():
    return pl.kernel(
        _sc_gather_body,
        mesh=plsc.VectorSubcoreMesh(core_axis_name="c", subcore_axis_name="s"),
        out_type=[
            jax.ShapeDtypeStruct((BATCH, URL_DIM), jnp.float32),
            jax.ShapeDtypeStruct((BATCH, AUTHOR_DIM), jnp.float32),
        ],
        scratch_types=[
            pltpu.VMEM((_ROWS_PER_TILE,), jnp.int32),
            pltpu.VMEM((_ROWS_PER_TILE,), jnp.int32),
            pltpu.VMEM((_ROWS_PER_TILE, URL_DIM), jnp.float32),
            pltpu.VMEM((_ROWS_PER_TILE, AUTHOR_DIM), jnp.float32),
            pltpu.SemaphoreType.DMA,
        ],
        compiler_params=pltpu.CompilerParams(use_tc_tiling_on_sc=False),
    )


_BM = 4096


def _mlp_body(title, urle, authe, w1t, w1u, w1a, b1, w2, b2, w3, b3, out):
    h1 = (jnp.dot(title[:], w1t[:], preferred_element_type=jnp.float32)
          + jnp.dot(urle[:], w1u[:], preferred_element_type=jnp.float32)
          + jnp.dot(authe[:], w1a[:], preferred_element_type=jnp.float32)
          + b1[:])
    h1 = jnp.maximum(h1, 0.0)
    h2 = jnp.maximum(jnp.dot(h1, w2[:], preferred_element_type=jnp.float32) + b2[:], 0.0)
    out[:] = jnp.sum(h2 * w3[:].reshape(1, 64), axis=1, keepdims=True) + b3[:]


def _mlp(title_emb, url_emb, author_emb, W1t, W1u, W1a, b1, W2, b2, W3, b3):
    grid = (BATCH // _BM,)
    return pl.pallas_call(
        _mlp_body,
        grid=grid,
        in_specs=[
            pl.BlockSpec((_BM, WORD_DIM), lambda i: (i, 0)),
            pl.BlockSpec((_BM, URL_DIM), lambda i: (i, 0)),
            pl.BlockSpec((_BM, AUTHOR_DIM), lambda i: (i, 0)),
            pl.BlockSpec((WORD_DIM, 128), lambda i: (0, 0)),
            pl.BlockSpec((URL_DIM, 128), lambda i: (0, 0)),
            pl.BlockSpec((AUTHOR_DIM, 128), lambda i: (0, 0)),
            pl.BlockSpec((1, 128), lambda i: (0, 0)),
            pl.BlockSpec((128, 64), lambda i: (0, 0)),
            pl.BlockSpec((1, 64), lambda i: (0, 0)),
            pl.BlockSpec((64, 1), lambda i: (0, 0)),
            pl.BlockSpec((1, 1), lambda i: (0, 0)),
        ],
        out_specs=pl.BlockSpec((_BM, 1), lambda i: (i, 0)),
        out_shape=jax.ShapeDtypeStruct((BATCH, 1), jnp.float32),
    )(title_emb, url_emb, author_emb, W1t, W1u, W1a, b1, W2, b2, W3, b3)


def kernel(title_emb, url_idx, author_idx, url_table, author_table, W1, b1, W2, b2, W3, b3):
    url_idx = url_idx.astype(jnp.int32)
    author_idx = author_idx.astype(jnp.int32)
    url_emb, author_emb = _sc_gather()(url_table, author_table, url_idx, author_idx)
    W1t = W1[:WORD_DIM]
    W1u = W1[WORD_DIM:WORD_DIM + URL_DIM]
    W1a = W1[WORD_DIM + URL_DIM:]
    return _mlp(title_emb, url_emb, author_emb,
                W1t, W1u, W1a,
                b1.reshape(1, 128), W2, b2.reshape(1, 64),
                W3, b3.reshape(1, 1))
